# baseline (speedup 1.0000x reference)
import functools

import jax
import jax.numpy as jnp
from jax import lax
from jax.experimental import pallas as pl
from jax.experimental.pallas import tpu as pltpu
from jax.experimental.pallas import tpu_sc as plsc

_NW = 32


def _proj_t_body(p1_ref, p2_ref, proj_ref, out_ref):
    proj = proj_ref[...]
    pn = proj * jax.lax.rsqrt(jnp.sum(proj * proj, axis=0, keepdims=True))
    s1 = 1.0 / (1.0 + jnp.exp(-p1_ref[...]))
    s2 = 1.0 / (1.0 + jnp.exp(-p2_ref[...]))
    dn = (((0,), (1,)), ((), ()))
    z1t = jax.lax.dot_general(pn, s1, dn, precision=jax.lax.Precision.HIGHEST)
    z2t = jax.lax.dot_general(pn, s2, dn, precision=jax.lax.Precision.HIGHEST)
    out_ref[...] = jnp.concatenate([z1t, z2t], axis=0)


def _sc_sort_stage(zt, n, m):
    pairs_per_w = m // _NW
    seg = n // 16
    mesh = plsc.VectorSubcoreMesh(core_axis_name="c", subcore_axis_name="s")

    @functools.partial(
        pl.kernel,
        mesh=mesh,
        out_type=jax.ShapeDtypeStruct((_NW, 16), jnp.float32),
        compiler_params=pltpu.CompilerParams(needs_layout_passes=False),
        scratch_types=[
            pltpu.VMEM((n,), jnp.float32),
            pltpu.VMEM((n,), jnp.int32),
            pltpu.VMEM((n,), jnp.int32),
            pltpu.VMEM((n,), jnp.float32),
            pltpu.VMEM((4096,), jnp.int32),
            pltpu.VMEM((16,), jnp.float32),
        ],
    )
    def body(zt_hbm, out_hbm, f_v, k0_v, k1_v, a_v, hist_v, acc_v):
        w = lax.axis_index("s") * 2 + lax.axis_index("c")
        lane = lax.iota(jnp.int32, 16)
        gidx = lane * seg
        ones = jnp.ones((16,), jnp.int32)
        minint = jnp.int32(-2147483648)

        def to_keys(i, carry):
            v = lax.bitcast_convert_type(f_v[pl.ds(i * 16, 16)], jnp.int32)
            mask = lax.shift_right_arithmetic(v, 31)
            k0_v[pl.ds(i * 16, 16)] = v ^ (mask | minint)
            return carry

        def radix_pass(src_ref, dst_ref, shift):
            def zero_it(i, carry):
                hist_v[pl.ds(i * 16, 16)] = jnp.zeros((16,), jnp.int32)
                return carry

            lax.fori_loop(0, 256, zero_it, 0)

            def hist_it(i, carry):
                v = plsc.load_gather(src_ref, [gidx + i])
                dig = lax.shift_right_logical(v, shift) & 255
                plsc.addupdate_scatter(hist_v, [dig * 16 + lane], ones)
                return carry

            lax.fori_loop(0, seg, hist_it, 0)

            def scan_it(d, carry):
                h = hist_v[pl.ds(d * 16, 16)]
                hist_v[pl.ds(d * 16, 16)] = plsc.cumsum(h) - h + carry
                return carry + jnp.sum(h)

            lax.fori_loop(0, 256, scan_it, jnp.int32(0))

            def perm_it(i, carry):
                v = plsc.load_gather(src_ref, [gidx + i])
                dig = lax.shift_right_logical(v, shift) & 255
                slot = dig * 16 + lane
                pos = plsc.load_gather(hist_v, [slot])
                plsc.store_scatter(dst_ref, [pos], v)
                plsc.addupdate_scatter(hist_v, [slot], ones)
                return carry

            lax.fori_loop(0, seg, perm_it, 0)

        def sort_col(col):
            pltpu.sync_copy(zt_hbm.at[col], f_v)
            lax.fori_loop(0, n // 16, to_keys, 0)
            radix_pass(k0_v, k1_v, 0)
            radix_pass(k1_v, k0_v, 8)
            radix_pass(k0_v, k1_v, 16)
            radix_pass(k1_v, k0_v, 24)

        def inv(u):
            mask = lax.shift_right_arithmetic(u, 31)
            return lax.bitcast_convert_type(u ^ (jnp.invert(mask) | minint), jnp.float32)

        acc_v[...] = jnp.zeros((16,), jnp.float32)

        for q in range(pairs_per_w):
            col = w * pairs_per_w + q
            sort_col(col)

            def store_a(i, carry):
                a_v[pl.ds(i * 16, 16)] = inv(k0_v[pl.ds(i * 16, 16)])
                return carry

            lax.fori_loop(0, n // 16, store_a, 0)

            sort_col(col + m)

            def diff_it(i, carry):
                d = a_v[pl.ds(i * 16, 16)] - inv(k0_v[pl.ds(i * 16, 16)])
                acc_v[...] = acc_v[...] + d * d
                return carry

            lax.fori_loop(0, n // 16, diff_it, 0)

        pltpu.sync_copy(acc_v, out_hbm.at[w])

    return body(zt)


def kernel(p1, p2, proj):
    n, c = p1.shape
    m = proj.shape[1]
    row_blk = 2048

    zt = pl.pallas_call(
        _proj_t_body,
        grid=(n // row_blk,),
        in_specs=[
            pl.BlockSpec((row_blk, c), lambda i: (i, 0)),
            pl.BlockSpec((row_blk, c), lambda i: (i, 0)),
            pl.BlockSpec((c, m), lambda i: (0, 0)),
        ],
        out_specs=pl.BlockSpec((2 * m, row_blk), lambda i: (0, i)),
        out_shape=jax.ShapeDtypeStruct((2 * m, n), jnp.float32),
    )(p1, p2, proj)

    parts = _sc_sort_stage(zt, n, m)
    return jnp.sum(parts) / jnp.float32(n * m)


# speedup vs baseline: 1.2764x; 1.2764x over previous
import functools

import jax
import jax.numpy as jnp
from jax import lax
from jax.experimental import pallas as pl
from jax.experimental.pallas import tpu as pltpu
from jax.experimental.pallas import tpu_sc as plsc

_NW = 32


def _proj_t_body(p1_ref, p2_ref, proj_ref, out_ref):
    proj = proj_ref[...]
    pn = proj * jax.lax.rsqrt(jnp.sum(proj * proj, axis=0, keepdims=True))
    s1 = 1.0 / (1.0 + jnp.exp(-p1_ref[...]))
    s2 = 1.0 / (1.0 + jnp.exp(-p2_ref[...]))
    dn = (((0,), (1,)), ((), ()))
    z1t = jax.lax.dot_general(pn, s1, dn, precision=jax.lax.Precision.HIGHEST)
    z2t = jax.lax.dot_general(pn, s2, dn, precision=jax.lax.Precision.HIGHEST)
    out_ref[...] = jnp.concatenate([z1t, z2t], axis=0)


def _sc_sort_stage(zt, n, m):
    pairs_per_w = m // _NW
    seg = n // 16
    mesh = plsc.VectorSubcoreMesh(core_axis_name="c", subcore_axis_name="s")

    @functools.partial(
        pl.kernel,
        mesh=mesh,
        out_type=jax.ShapeDtypeStruct((_NW, 16), jnp.float32),
        compiler_params=pltpu.CompilerParams(needs_layout_passes=False),
        scratch_types=[
            pltpu.VMEM((n,), jnp.float32),
            pltpu.VMEM((n,), jnp.float32),
            pltpu.VMEM((n,), jnp.int32),
            pltpu.VMEM((n,), jnp.int32),
            pltpu.VMEM((n,), jnp.int32),
            pltpu.VMEM((4096,), jnp.int32),
            pltpu.VMEM((4096,), jnp.int32),
            pltpu.VMEM((16,), jnp.float32),
            pltpu.SemaphoreType.DMA,
            pltpu.SemaphoreType.DMA,
        ],
    )
    def body(zt_hbm, out_hbm, f_v, g_v, k0_v, k1_v, ka_v, ha_v, hb_v, acc_v,
             sema, semb):
        w = lax.axis_index("s") * 2 + lax.axis_index("c")
        lane = lax.iota(jnp.int32, 16)
        seg2 = n // 32
        base_a = lane * seg2
        base_b = base_a + n // 2
        ones = jnp.ones((16,), jnp.int32)
        zeros = jnp.zeros((16,), jnp.int32)
        minint = jnp.int32(-2147483648)

        def xform(v):
            b = lax.bitcast_convert_type(v, jnp.int32)
            mask = lax.shift_right_arithmetic(b, 31)
            return b ^ (mask | minint)

        def radix_pass(src_ref, dst_ref, shift, first):
            def zero_it(i, carry):
                ha_v[pl.ds(i * 16, 16)] = zeros
                hb_v[pl.ds(i * 16, 16)] = zeros
                return carry

            lax.fori_loop(0, 256, zero_it, 0)

            def hist_it(i, carry):
                va = plsc.load_gather(src_ref, [base_a + i])
                vb = plsc.load_gather(src_ref, [base_b + i])
                if first:
                    va = xform(va)
                    vb = xform(vb)
                da = lax.shift_right_logical(va, shift) & 255
                db = lax.shift_right_logical(vb, shift) & 255
                plsc.addupdate_scatter(ha_v, [da * 16 + lane], ones)
                plsc.addupdate_scatter(hb_v, [db * 16 + lane], ones)
                return carry

            lax.fori_loop(0, seg2, hist_it, 0)

            def scan_it(d, carry):
                ha = ha_v[pl.ds(d * 16, 16)]
                hb = hb_v[pl.ds(d * 16, 16)]
                sa = jnp.sum(ha)
                ha_v[pl.ds(d * 16, 16)] = plsc.cumsum(ha) - ha + carry
                hb_v[pl.ds(d * 16, 16)] = plsc.cumsum(hb) - hb + (carry + sa)
                return carry + sa + jnp.sum(hb)

            lax.fori_loop(0, 256, scan_it, jnp.int32(0))

            def perm_it(i, carry):
                va = plsc.load_gather(src_ref, [base_a + i])
                vb = plsc.load_gather(src_ref, [base_b + i])
                if first:
                    va = xform(va)
                    vb = xform(vb)
                da = lax.shift_right_logical(va, shift) & 255
                db = lax.shift_right_logical(vb, shift) & 255
                slta = da * 16 + lane
                sltb = db * 16 + lane
                pa = plsc.load_gather(ha_v, [slta])
                pb = plsc.load_gather(hb_v, [sltb])
                plsc.store_scatter(dst_ref, [pa], va)
                plsc.store_scatter(dst_ref, [pb], vb)
                plsc.addupdate_scatter(ha_v, [slta], ones)
                plsc.addupdate_scatter(hb_v, [sltb], ones)
                return carry

            lax.fori_loop(0, seg2, perm_it, 0)

        def sort_col(src_ref, dst_ref):
            radix_pass(src_ref, k0_v, 0, True)
            radix_pass(k0_v, k1_v, 8, False)
            radix_pass(k1_v, k0_v, 16, False)
            radix_pass(k0_v, dst_ref, 24, False)

        def inv(u):
            mask = lax.shift_right_arithmetic(u, 31)
            return lax.bitcast_convert_type(
                u ^ (jnp.invert(mask) | minint), jnp.float32
            )

        acc = jnp.zeros((16,), jnp.float32)
        pltpu.async_copy(zt_hbm.at[w * pairs_per_w], f_v, sema).wait()

        for q in range(pairs_per_w):
            col = w * pairs_per_w + q
            cpb = pltpu.async_copy(zt_hbm.at[col + m], g_v, semb)
            sort_col(f_v, ka_v)
            cpb.wait()
            if q + 1 < pairs_per_w:
                cpa = pltpu.async_copy(zt_hbm.at[col + 1], f_v, sema)
            sort_col(g_v, k1_v)

            def diff_it(i, acc):
                d1 = inv(ka_v[pl.ds(i * 16, 16)]) - inv(k1_v[pl.ds(i * 16, 16)])
                j = i + n // 32
                d2 = inv(ka_v[pl.ds(j * 16, 16)]) - inv(k1_v[pl.ds(j * 16, 16)])
                return acc + d1 * d1 + d2 * d2

            acc = lax.fori_loop(0, n // 32, diff_it, acc)
            if q + 1 < pairs_per_w:
                cpa.wait()

        acc_v[...] = acc
        pltpu.sync_copy(acc_v, out_hbm.at[w])

    return body(zt)


def kernel(p1, p2, proj):
    n, c = p1.shape
    m = proj.shape[1]
    row_blk = 2048

    zt = pl.pallas_call(
        _proj_t_body,
        grid=(n // row_blk,),
        in_specs=[
            pl.BlockSpec((row_blk, c), lambda i: (i, 0)),
            pl.BlockSpec((row_blk, c), lambda i: (i, 0)),
            pl.BlockSpec((c, m), lambda i: (0, 0)),
        ],
        out_specs=pl.BlockSpec((2 * m, row_blk), lambda i: (0, i)),
        out_shape=jax.ShapeDtypeStruct((2 * m, n), jnp.float32),
    )(p1, p2, proj)

    parts = _sc_sort_stage(zt, n, m)
    return jnp.sum(parts) / jnp.float32(n * m)


# speedup vs baseline: 1.4369x; 1.1258x over previous
import functools

import jax
import jax.numpy as jnp
from jax import lax
from jax.experimental import pallas as pl
from jax.experimental.pallas import tpu as pltpu
from jax.experimental.pallas import tpu_sc as plsc

_NW = 32


def _proj_t_body(p1_ref, p2_ref, proj_ref, out_ref):
    proj = proj_ref[...]
    pn = proj * jax.lax.rsqrt(jnp.sum(proj * proj, axis=0, keepdims=True))
    s1 = 1.0 / (1.0 + jnp.exp(-p1_ref[...]))
    s2 = 1.0 / (1.0 + jnp.exp(-p2_ref[...]))
    dn = (((0,), (1,)), ((), ()))
    z1t = jax.lax.dot_general(pn, s1, dn, precision=jax.lax.Precision.HIGHEST)
    z2t = jax.lax.dot_general(pn, s2, dn, precision=jax.lax.Precision.HIGHEST)
    out_ref[...] = jnp.concatenate([z1t, z2t], axis=0)


def _sc_sort_stage(zt, n, m):
    pairs_per_w = m // _NW
    seg = n // 16
    mesh = plsc.VectorSubcoreMesh(core_axis_name="c", subcore_axis_name="s")

    @functools.partial(
        pl.kernel,
        mesh=mesh,
        out_type=jax.ShapeDtypeStruct((_NW, 16), jnp.float32),
        compiler_params=pltpu.CompilerParams(needs_layout_passes=False),
        scratch_types=[
            pltpu.VMEM((n,), jnp.float32),
            pltpu.VMEM((n,), jnp.float32),
            pltpu.VMEM((n,), jnp.int32),
            pltpu.VMEM((n,), jnp.int32),
            pltpu.VMEM((n,), jnp.int32),
            pltpu.VMEM((4096,), jnp.int32),
            pltpu.VMEM((4096,), jnp.int32),
            pltpu.VMEM((4096,), jnp.int32),
            pltpu.VMEM((4096,), jnp.int32),
            pltpu.VMEM((16,), jnp.float32),
            pltpu.SemaphoreType.DMA,
            pltpu.SemaphoreType.DMA,
        ],
    )
    def body(zt_hbm, out_hbm, f_v, g_v, k0_v, k1_v, ka_v, h0_v, h1_v, h2_v,
             h3_v, acc_v, sema, semb):
        w = lax.axis_index("s") * 2 + lax.axis_index("c")
        lane = lax.iota(jnp.int32, 16)
        hists = [h0_v, h1_v, h2_v, h3_v]
        nc = len(hists)
        seg2 = n // (16 * nc)
        bases = [lane * seg2 + c * (n // nc) for c in range(nc)]
        ones = jnp.ones((16,), jnp.int32)
        zeros = jnp.zeros((16,), jnp.int32)
        minint = jnp.int32(-2147483648)

        def xform(v):
            b = lax.bitcast_convert_type(v, jnp.int32)
            mask = lax.shift_right_arithmetic(b, 31)
            return b ^ (mask | minint)

        def radix_pass(src_ref, dst_ref, shift, first):
            def zero_it(i, carry):
                for h in hists:
                    h[pl.ds(i * 16, 16)] = zeros
                return carry

            lax.fori_loop(0, 256, zero_it, 0)

            def digits(i):
                out = []
                for c in range(nc):
                    v = plsc.load_gather(src_ref, [bases[c] + i])
                    if first:
                        v = xform(v)
                    out.append((v, lax.shift_right_logical(v, shift) & 255))
                return out

            def hist_it(i, carry):
                for c, (_, d) in enumerate(digits(i)):
                    plsc.addupdate_scatter(hists[c], [d * 16 + lane], ones)
                return carry

            lax.fori_loop(0, seg2, hist_it, 0)

            def scan_it(d, carry):
                hs = [h[pl.ds(d * 16, 16)] for h in hists]
                for c, h in enumerate(hs):
                    hists[c][pl.ds(d * 16, 16)] = plsc.cumsum(h) - h + carry
                    carry = carry + jnp.sum(h)
                return carry

            lax.fori_loop(0, 256, scan_it, jnp.int32(0))

            def perm_it(i, carry):
                dv = digits(i)
                slots = [d * 16 + lane for _, d in dv]
                pos = [plsc.load_gather(hists[c], [slots[c]]) for c in range(nc)]
                for c, (v, _) in enumerate(dv):
                    plsc.store_scatter(dst_ref, [pos[c]], v)
                    plsc.addupdate_scatter(hists[c], [slots[c]], ones)
                return carry

            lax.fori_loop(0, seg2, perm_it, 0)

        def sort_col(src_ref, dst_ref):
            radix_pass(src_ref, k0_v, 0, True)
            radix_pass(k0_v, k1_v, 8, False)
            radix_pass(k1_v, k0_v, 16, False)
            radix_pass(k0_v, dst_ref, 24, False)

        def inv(u):
            mask = lax.shift_right_arithmetic(u, 31)
            return lax.bitcast_convert_type(
                u ^ (jnp.invert(mask) | minint), jnp.float32
            )

        acc = jnp.zeros((16,), jnp.float32)
        pltpu.async_copy(zt_hbm.at[w * pairs_per_w], f_v, sema).wait()

        for q in range(pairs_per_w):
            col = w * pairs_per_w + q
            cpb = pltpu.async_copy(zt_hbm.at[col + m], g_v, semb)
            sort_col(f_v, ka_v)
            cpb.wait()
            if q + 1 < pairs_per_w:
                cpa = pltpu.async_copy(zt_hbm.at[col + 1], f_v, sema)
            sort_col(g_v, k1_v)

            def diff_it(i, acc):
                for c in range(nc):
                    j = i + c * (n // (16 * nc))
                    d = inv(ka_v[pl.ds(j * 16, 16)]) - inv(k1_v[pl.ds(j * 16, 16)])
                    acc = acc + d * d
                return acc

            acc = lax.fori_loop(0, n // (16 * nc), diff_it, acc)
            if q + 1 < pairs_per_w:
                cpa.wait()

        acc_v[...] = acc
        pltpu.sync_copy(acc_v, out_hbm.at[w])

    return body(zt)


def kernel(p1, p2, proj):
    n, c = p1.shape
    m = proj.shape[1]
    row_blk = 2048

    zt = pl.pallas_call(
        _proj_t_body,
        grid=(n // row_blk,),
        in_specs=[
            pl.BlockSpec((row_blk, c), lambda i: (i, 0)),
            pl.BlockSpec((row_blk, c), lambda i: (i, 0)),
            pl.BlockSpec((c, m), lambda i: (0, 0)),
        ],
        out_specs=pl.BlockSpec((2 * m, row_blk), lambda i: (0, i)),
        out_shape=jax.ShapeDtypeStruct((2 * m, n), jnp.float32),
    )(p1, p2, proj)

    parts = _sc_sort_stage(zt, n, m)
    return jnp.sum(parts) / jnp.float32(n * m)


# speedup vs baseline: 2.4810x; 1.7266x over previous
import functools

import jax
import jax.numpy as jnp
from jax import lax
from jax.experimental import pallas as pl
from jax.experimental.pallas import tpu as pltpu
from jax.experimental.pallas import tpu_sc as plsc

_NW = 32


def _proj_split_body(p1_ref, p2_ref, proj_ref, zt_ref, z_ref):
    proj = proj_ref[...]
    m = proj.shape[1]
    msc = zt_ref.shape[0] // 2
    pn = proj * jax.lax.rsqrt(jnp.sum(proj * proj, axis=0, keepdims=True))
    s1 = 1.0 / (1.0 + jnp.exp(-p1_ref[...]))
    s2 = 1.0 / (1.0 + jnp.exp(-p2_ref[...]))
    hi = jax.lax.Precision.HIGHEST
    z1 = jax.lax.dot(s1, pn, precision=hi)
    z2 = jax.lax.dot(s2, pn, precision=hi)
    zt_ref[...] = jnp.concatenate(
        [z1[:, :msc].T, z2[:, :msc].T], axis=0
    )
    z_ref[...] = jnp.concatenate([z1[:, msc:], z2[:, msc:]], axis=1)


def _sc_sort_stage(zt, n, m):
    pairs_per_w = m // _NW
    seg = n // 16
    mesh = plsc.VectorSubcoreMesh(core_axis_name="c", subcore_axis_name="s")

    @functools.partial(
        pl.kernel,
        mesh=mesh,
        out_type=jax.ShapeDtypeStruct((_NW, 16), jnp.float32),
        compiler_params=pltpu.CompilerParams(needs_layout_passes=False),
        scratch_types=[
            pltpu.VMEM((n,), jnp.float32),
            pltpu.VMEM((n,), jnp.float32),
            pltpu.VMEM((n,), jnp.int32),
            pltpu.VMEM((n,), jnp.int32),
            pltpu.VMEM((n,), jnp.int32),
            pltpu.VMEM((4096,), jnp.int32),
            pltpu.VMEM((4096,), jnp.int32),
            pltpu.VMEM((4096,), jnp.int32),
            pltpu.VMEM((4096,), jnp.int32),
            pltpu.VMEM((16,), jnp.float32),
            pltpu.SemaphoreType.DMA,
            pltpu.SemaphoreType.DMA,
        ],
    )
    def body(zt_hbm, out_hbm, f_v, g_v, k0_v, k1_v, ka_v, h0_v, h1_v, h2_v,
             h3_v, acc_v, sema, semb):
        w = lax.axis_index("s") * 2 + lax.axis_index("c")
        lane = lax.iota(jnp.int32, 16)
        hists = [h0_v, h1_v, h2_v, h3_v]
        nc = len(hists)
        seg2 = n // (16 * nc)
        bases = [lane * seg2 + c * (n // nc) for c in range(nc)]
        ones = jnp.ones((16,), jnp.int32)
        zeros = jnp.zeros((16,), jnp.int32)
        minint = jnp.int32(-2147483648)

        def xform(v):
            b = lax.bitcast_convert_type(v, jnp.int32)
            mask = lax.shift_right_arithmetic(b, 31)
            return b ^ (mask | minint)

        def radix_pass(src_ref, dst_ref, shift, first):
            def zero_it(i, carry):
                for h in hists:
                    h[pl.ds(i * 16, 16)] = zeros
                return carry

            lax.fori_loop(0, 256, zero_it, 0)

            def digits(i):
                out = []
                for c in range(nc):
                    v = plsc.load_gather(src_ref, [bases[c] + i])
                    if first:
                        v = xform(v)
                    out.append((v, lax.shift_right_logical(v, shift) & 255))
                return out

            def hist_it(i, carry):
                for c, (_, d) in enumerate(digits(i)):
                    plsc.addupdate_scatter(hists[c], [d * 16 + lane], ones)
                return carry

            lax.fori_loop(0, seg2, hist_it, 0)

            def scan_it(d, carry):
                hs = [h[pl.ds(d * 16, 16)] for h in hists]
                for c, h in enumerate(hs):
                    hists[c][pl.ds(d * 16, 16)] = plsc.cumsum(h) - h + carry
                    carry = carry + jnp.sum(h)
                return carry

            lax.fori_loop(0, 256, scan_it, jnp.int32(0))

            def perm_it(i, carry):
                dv = digits(i)
                slots = [d * 16 + lane for _, d in dv]
                pos = [plsc.load_gather(hists[c], [slots[c]]) for c in range(nc)]
                for c, (v, _) in enumerate(dv):
                    plsc.store_scatter(dst_ref, [pos[c]], v)
                    plsc.addupdate_scatter(hists[c], [slots[c]], ones)
                return carry

            lax.fori_loop(0, seg2, perm_it, 0)

        def sort_col(src_ref, dst_ref):
            radix_pass(src_ref, k0_v, 0, True)
            radix_pass(k0_v, k1_v, 8, False)
            radix_pass(k1_v, k0_v, 16, False)
            radix_pass(k0_v, dst_ref, 24, False)

        def inv(u):
            mask = lax.shift_right_arithmetic(u, 31)
            return lax.bitcast_convert_type(
                u ^ (jnp.invert(mask) | minint), jnp.float32
            )

        acc = jnp.zeros((16,), jnp.float32)
        pltpu.async_copy(zt_hbm.at[w * pairs_per_w], f_v, sema).wait()

        for q in range(pairs_per_w):
            col = w * pairs_per_w + q
            cpb = pltpu.async_copy(zt_hbm.at[col + m], g_v, semb)
            sort_col(f_v, ka_v)
            cpb.wait()
            if q + 1 < pairs_per_w:
                cpa = pltpu.async_copy(zt_hbm.at[col + 1], f_v, sema)
            sort_col(g_v, k1_v)

            def diff_it(i, acc):
                for c in range(nc):
                    j = i + c * (n // (16 * nc))
                    d = inv(ka_v[pl.ds(j * 16, 16)]) - inv(k1_v[pl.ds(j * 16, 16)])
                    acc = acc + d * d
                return acc

            acc = lax.fori_loop(0, n // (16 * nc), diff_it, acc)
            if q + 1 < pairs_per_w:
                cpa.wait()

        acc_v[...] = acc
        pltpu.sync_copy(acc_v, out_hbm.at[w])

    return body(zt)


_CHUNK = 512


def _proj_body(p1_ref, p2_ref, proj_ref, out_ref):
    proj = proj_ref[...]
    pn = proj * jax.lax.rsqrt(jnp.sum(proj * proj, axis=0, keepdims=True))
    s1 = 1.0 / (1.0 + jnp.exp(-p1_ref[...]))
    s2 = 1.0 / (1.0 + jnp.exp(-p2_ref[...]))
    z1 = jax.lax.dot(s1, pn, precision=jax.lax.Precision.HIGHEST)
    z2 = jax.lax.dot(s2, pn, precision=jax.lax.Precision.HIGHEST)
    out_ref[...] = jnp.concatenate([z1, z2], axis=1)


def _substage(xc, bk, bj, base):
    cc, _ = xc.shape
    d = 1 << bj
    i = jax.lax.broadcasted_iota(jnp.int32, (cc, 1), 0) + base
    is_lo = ((i >> bj) & 1) == 0
    partner = jnp.where(is_lo, jnp.roll(xc, -d, axis=0), jnp.roll(xc, d, axis=0))
    mn = jnp.minimum(xc, partner)
    mx = jnp.maximum(xc, partner)
    asc = ((i >> bk) & 1) == 0
    return jnp.where(asc == is_lo, mn, mx)


def _chunk_sort_body(z_ref, out_ref):
    cc = z_ref.shape[0]
    clog = cc.bit_length() - 1
    base = pl.program_id(0) * cc
    xc = z_ref[...]
    for bk in range(1, clog + 1):
        for bj in range(bk - 1, -1, -1):
            xc = _substage(xc, bk, bj, base)
    out_ref[...] = xc


def _merge_body(x_ref, out_ref):
    n, l = x_ref.shape
    m = l // 2
    cc = _CHUNK
    nch = n // cc
    nlog = n.bit_length() - 1
    clog = cc.bit_length() - 1

    for bk in range(clog + 1, nlog + 1):
        for bj in range(bk - 1, clog - 1, -1):
            dc = (1 << bj) // cc

            def cross(p, carry, bk=bk, dc=dc):
                c_lo = (p // dc) * 2 * dc + (p % dc)
                lo = x_ref[pl.ds(c_lo * cc, cc), :]
                hi = x_ref[pl.ds((c_lo + dc) * cc, cc), :]
                mn = jnp.minimum(lo, hi)
                mx = jnp.maximum(lo, hi)
                asc = ((c_lo * cc >> bk) & 1) == 0
                x_ref[pl.ds(c_lo * cc, cc), :] = jnp.where(asc, mn, mx)
                x_ref[pl.ds((c_lo + dc) * cc, cc), :] = jnp.where(asc, mx, mn)
                return carry

            jax.lax.fori_loop(0, nch // 2, cross, 0)

        def tail(c, carry, bk=bk):
            base = c * cc
            xc = x_ref[pl.ds(base, cc), :]
            for bj in range(clog - 1, -1, -1):
                xc = _substage(xc, bk, bj, base)
            x_ref[pl.ds(base, cc), :] = xc
            return carry

        jax.lax.fori_loop(0, nch, tail, 0)

    def reduce_body(c, acc):
        xc = x_ref[pl.ds(c * cc, cc), :]
        diff = xc[:, :m] - xc[:, m:]
        return acc + jnp.sum(diff * diff)

    out_ref[0, 0] = jax.lax.fori_loop(0, nch, reduce_body, jnp.float32(0.0))


def kernel(p1, p2, proj):
    n, c = p1.shape
    m = proj.shape[1]
    msc = m // 2
    mtc = m - msc
    row_blk = 2048

    zt, z = pl.pallas_call(
        _proj_split_body,
        grid=(n // row_blk,),
        in_specs=[
            pl.BlockSpec((row_blk, c), lambda i: (i, 0)),
            pl.BlockSpec((row_blk, c), lambda i: (i, 0)),
            pl.BlockSpec((c, m), lambda i: (0, 0)),
        ],
        out_specs=[
            pl.BlockSpec((2 * msc, row_blk), lambda i: (0, i)),
            pl.BlockSpec((row_blk, 2 * mtc), lambda i: (i, 0)),
        ],
        out_shape=[
            jax.ShapeDtypeStruct((2 * msc, n), jnp.float32),
            jax.ShapeDtypeStruct((n, 2 * mtc), jnp.float32),
        ],
    )(p1, p2, proj)

    parts = _sc_sort_stage(zt, n, msc)

    zs = pl.pallas_call(
        _chunk_sort_body,
        grid=(n // _CHUNK,),
        in_specs=[pl.BlockSpec((_CHUNK, 2 * mtc), lambda i: (i, 0))],
        out_specs=pl.BlockSpec((_CHUNK, 2 * mtc), lambda i: (i, 0)),
        out_shape=jax.ShapeDtypeStruct((n, 2 * mtc), jnp.float32),
    )(z)

    ssq_tc = pl.pallas_call(
        _merge_body,
        in_specs=[pl.BlockSpec((n, 2 * mtc), lambda: (0, 0))],
        out_specs=pl.BlockSpec(memory_space=pltpu.SMEM),
        out_shape=jax.ShapeDtypeStruct((1, 1), jnp.float32),
    )(zs)

    return (jnp.sum(parts) + ssq_tc[0, 0]) / jnp.float32(n * m)
